# Initial kernel scaffold; baseline (speedup 1.0000x reference)
import jax
import jax.numpy as jnp
from jax.experimental import pallas as pl


def kernel(positions, hidden_states, ln1_w, wq, wk, wv, sinks, wo, ln2_w, router_w, router_b, w1, b1, w2, b2):
    raise NotImplementedError("write your pallas kernel here")


# speedup vs baseline: 1.7802x; 1.7802x over previous
import functools

import jax
import jax.numpy as jnp
from jax import lax
from jax.experimental import pallas as pl
from jax.experimental.pallas import tpu as pltpu
from jax.experimental.pallas import tpu_sc as plsc

T = 2048; D = 2048; H = 16; HKV = 8; DH = 128; E = 8; F = 2048
ALPHA = 1.702; LIMIT = 7.0; EPS = 1e-6; SCALE = DH ** -0.5
THETA = 10000.0

BT = 256
BD = 128
NB = 40
NPAD = NB * BD
NBPAD = 64

f32 = jnp.float32
bf16 = jnp.bfloat16


def _qkv_body(x_ref, w_ref, ln_ref, cos_ref, sin_ref, o_ref):
    j = pl.program_id(1)
    x = x_ref[...]
    xn = x * lax.rsqrt(jnp.mean(x * x, axis=1, keepdims=True) + EPS)
    xn = xn * ln_ref[...]
    y = jnp.dot(xn.astype(bf16), w_ref[...], preferred_element_type=f32)
    swap = jnp.concatenate([y[:, DH // 2:], y[:, :DH // 2]], axis=1)
    roped = y * cos_ref[...] + swap * sin_ref[...]
    out = jnp.where(j < H + HKV, roped, y)
    o_ref[...] = out[None]


def _qkv(hidden, wqkv, ln1, cos_t, sin_t):
    nslots = H + 2 * HKV
    return pl.pallas_call(
        _qkv_body,
        grid=(T // BT, nslots),
        in_specs=[
            pl.BlockSpec((BT, D), lambda i, j: (i, 0)),
            pl.BlockSpec((D, DH), lambda i, j: (0, j)),
            pl.BlockSpec((1, D), lambda i, j: (0, 0)),
            pl.BlockSpec((BT, DH), lambda i, j: (i, 0)),
            pl.BlockSpec((BT, DH), lambda i, j: (i, 0)),
        ],
        out_specs=pl.BlockSpec((1, BT, DH), lambda i, j: (j, i, 0)),
        out_shape=jax.ShapeDtypeStruct((nslots, T, DH), f32),
    )(hidden, wqkv, ln1, cos_t, sin_t)


def _attn_body(q_ref, k_ref, v_ref, sinks_ref, o_ref):
    h = pl.program_id(0)
    i = pl.program_id(1)
    q = q_ref[0].astype(bf16)
    k = k_ref[0].astype(bf16)
    v = v_ref[0].astype(bf16)
    s = lax.dot_general(q, k, (((1,), (1,)), ((), ())),
                        preferred_element_type=f32) * SCALE
    row = lax.broadcasted_iota(jnp.int32, (BT, T), 0) + i * BT
    col = lax.broadcasted_iota(jnp.int32, (BT, T), 1)
    s = jnp.where(col <= row, s, -1e30)
    snk = sinks_ref[h]
    m = jnp.maximum(jnp.max(s, axis=1, keepdims=True), snk)
    p = jnp.exp(s - m)
    l = jnp.sum(p, axis=1, keepdims=True) + jnp.exp(snk - m)
    o_ref[...] = jnp.dot((p / l).astype(bf16), v,
                         preferred_element_type=f32)


def _attention(qkv, sinks):
    return pl.pallas_call(
        _attn_body,
        grid=(H, T // BT),
        in_specs=[
            pl.BlockSpec((1, BT, DH), lambda h, i: (h, i, 0)),
            pl.BlockSpec((1, T, DH), lambda h, i: (H + h // 2, 0, 0)),
            pl.BlockSpec((1, T, DH), lambda h, i: (H + HKV + h // 2, 0, 0)),
            pl.BlockSpec(memory_space=pltpu.SMEM),
        ],
        out_specs=pl.BlockSpec((BT, DH), lambda h, i: (i, h)),
        out_shape=jax.ShapeDtypeStruct((T, H * DH), f32),
    )(qkv, qkv, qkv, sinks)


def _oproj_body(attn_ref, wo_ref, hid_ref, ln2_ref, rw_ref, rb_ref,
                h_ref, x2_ref, lg_ref):
    h = hid_ref[...] + jnp.dot(attn_ref[...].astype(bf16), wo_ref[...],
                               preferred_element_type=f32)
    h_ref[...] = h
    x2 = h * lax.rsqrt(jnp.mean(h * h, axis=1, keepdims=True) + EPS)
    x2 = x2 * ln2_ref[...]
    x2_ref[...] = x2
    lg_ref[...] = jnp.dot(x2.astype(bf16), rw_ref[...],
                          preferred_element_type=f32) + rb_ref[...]


def _oproj_router(attn, wo, hidden, ln2, rw, rb):
    return pl.pallas_call(
        _oproj_body,
        grid=(T // BT,),
        in_specs=[
            pl.BlockSpec((BT, H * DH), lambda i: (i, 0)),
            pl.BlockSpec((H * DH, D), lambda i: (0, 0)),
            pl.BlockSpec((BT, D), lambda i: (i, 0)),
            pl.BlockSpec((1, D), lambda i: (0, 0)),
            pl.BlockSpec((D, E), lambda i: (0, 0)),
            pl.BlockSpec((1, E), lambda i: (0, 0)),
        ],
        out_specs=[
            pl.BlockSpec((BT, D), lambda i: (i, 0)),
            pl.BlockSpec((BT, D), lambda i: (i, 0)),
            pl.BlockSpec((BT, E), lambda i: (i, 0)),
        ],
        out_shape=[
            jax.ShapeDtypeStruct((T, D), f32),
            jax.ShapeDtypeStruct((T, D), f32),
            jax.ShapeDtypeStruct((T, E), f32),
        ],
    )(attn, wo, hidden, ln2, rw, rb)


def _route_body(lg_ref, dw_ref, meta_ref):
    lg = lg_ref[...]
    jl = lax.broadcasted_iota(jnp.int32, (T, E), 1).astype(f32)
    m1 = jnp.max(lg, axis=1, keepdims=True)
    i1 = jnp.min(jnp.where(lg == m1, jl, 1e9), axis=1, keepdims=True)
    masked = jnp.where(jl == i1, -jnp.inf, lg)
    m2 = jnp.max(masked, axis=1, keepdims=True)
    i2 = jnp.min(jnp.where(masked == m2, jl, 1e9), axis=1, keepdims=True)
    e2 = jnp.exp(m2 - m1)
    w0 = 1.0 / (1.0 + e2)
    w1 = e2 * w0

    oh0 = (jl == i1).astype(f32)
    oh1 = (jl == i2).astype(f32)
    s16 = jnp.concatenate([oh0, oh1], axis=1)
    c = s16
    sh = 1
    while sh < T:
        c = c + jnp.concatenate(
            [jnp.zeros((sh, 2 * E), f32), c[:T - sh]], axis=0)
        sh *= 2
    cexc = c - s16
    ctot = cexc[:, :E] + cexc[:, E:]
    cnt = jnp.sum(s16, axis=0, keepdims=True)
    cnt8 = cnt[:, :E] + cnt[:, E:]
    nb = jnp.floor((cnt8 + (BD - 1.0)) * (1.0 / BD))
    eidx = lax.broadcasted_iota(jnp.int32, (E, E), 0).astype(f32)
    eidx2 = lax.broadcasted_iota(jnp.int32, (E, E), 1).astype(f32)
    mle = (eidx <= eidx2).astype(f32)
    ends = jnp.dot(nb, mle, preferred_element_type=f32) * BD
    pad_off = ends - nb * BD
    rank0 = jnp.sum(oh0 * ctot, axis=1, keepdims=True)
    rank1 = jnp.sum(oh1 * ctot, axis=1, keepdims=True)
    d0 = rank0 + jnp.sum(oh0 * pad_off, axis=1, keepdims=True)
    d1 = rank1 + jnp.sum(oh1 * pad_off, axis=1, keepdims=True)
    dw_ref[...] = jnp.where(jl == 0, d0,
                  jnp.where(jl == 1, d1,
                  jnp.where(jl == 2, w0,
                  jnp.where(jl == 3, w1, 0.0))))

    bi = lax.broadcasted_iota(jnp.int32, (NBPAD, E), 0).astype(f32)
    jb = lax.broadcasted_iota(jnp.int32, (NBPAD, E), 1).astype(f32)
    be = jnp.sum((ends <= BD * bi).astype(f32), axis=1, keepdims=True)
    be = jnp.minimum(be, float(E - 1))
    valid = (BD * bi[:, :1] < ends[:, E - 1:]).astype(f32)
    meta_ref[...] = jnp.where(jb == 0, be, jnp.where(jb == 1, valid, 0.0))


def _route(logits):
    return pl.pallas_call(
        _route_body,
        out_shape=[
            jax.ShapeDtypeStruct((T, E), f32),
            jax.ShapeDtypeStruct((NBPAD, E), f32),
        ],
    )(logits)


_SC_MESH = dict(core_axis_name="c", subcore_axis_name="s")
_NW = 32
_CH = 32
_TPW = T // _NW


def _dispatch_scatter(x2, d0, d1):
    mesh = plsc.VectorSubcoreMesh(**_SC_MESH)

    @functools.partial(
        pl.kernel, mesh=mesh,
        out_type=jax.ShapeDtypeStruct((NPAD, D), f32),
        scratch_types=[
            pltpu.VMEM((_CH, D), f32),
            pltpu.VMEM((_CH,), jnp.int32),
            pltpu.VMEM((_CH,), jnp.int32),
            pltpu.SemaphoreType.DMA,
        ],
    )
    def k(x2_hbm, d0_hbm, d1_hbm, xs_hbm, rows_v, i0_v, i1_v, sem):
        wid = lax.axis_index("s") * 2 + lax.axis_index("c")
        for cnk in range(_TPW // _CH):
            base = wid * _TPW + cnk * _CH
            pltpu.sync_copy(d0_hbm.at[pl.ds(base, _CH)], i0_v)
            pltpu.sync_copy(d1_hbm.at[pl.ds(base, _CH)], i1_v)
            pltpu.sync_copy(x2_hbm.at[pl.ds(base, _CH)], rows_v)
            pltpu.async_copy(rows_v, xs_hbm.at[i0_v], sem).wait()
            pltpu.async_copy(rows_v, xs_hbm.at[i1_v], sem).wait()

    return k(x2, d0, d1)


def _combine_gather(y, d0, d1):
    mesh = plsc.VectorSubcoreMesh(**_SC_MESH)

    @functools.partial(
        pl.kernel, mesh=mesh,
        out_type=(jax.ShapeDtypeStruct((T, D), f32),
                  jax.ShapeDtypeStruct((T, D), f32)),
        scratch_types=[
            pltpu.VMEM((_CH, D), f32),
            pltpu.VMEM((_CH,), jnp.int32),
            pltpu.SemaphoreType.DMA,
        ],
    )
    def k(y_hbm, d0_hbm, d1_hbm, y0_hbm, y1_hbm, rows_v, idx_v, sem):
        wid = lax.axis_index("s") * 2 + lax.axis_index("c")
        for cnk in range(_TPW // _CH):
            base = wid * _TPW + cnk * _CH
            pltpu.sync_copy(d0_hbm.at[pl.ds(base, _CH)], idx_v)
            pltpu.async_copy(y_hbm.at[idx_v], rows_v, sem).wait()
            pltpu.sync_copy(rows_v, y0_hbm.at[pl.ds(base, _CH)])
            pltpu.sync_copy(d1_hbm.at[pl.ds(base, _CH)], idx_v)
            pltpu.async_copy(y_hbm.at[idx_v], rows_v, sem).wait()
            pltpu.sync_copy(rows_v, y1_hbm.at[pl.ds(base, _CH)])

    return k(y, d0, d1)


def _mlp1_body(be_ref, valid_ref, xs_ref, w_ref, b_ref, o_ref):
    b = pl.program_id(0)

    @pl.when(valid_ref[b] == 1)
    def _():
        xb = xs_ref[...].astype(bf16)
        gu = jnp.dot(xb, w_ref[0], preferred_element_type=f32)
        bb = b_ref[0]
        gate = jnp.minimum(gu[:, :F] + bb[:, :F], LIMIT)
        up = jnp.clip(gu[:, F:] + bb[:, F:], -LIMIT, LIMIT)
        glu = gate / (1.0 + jnp.exp(-ALPHA * gate))
        o_ref[...] = ((up + 1.0) * glu).astype(bf16)


def _mlp1(be, valid, xs, w1p, b1p):
    return pl.pallas_call(
        _mlp1_body,
        grid_spec=pltpu.PrefetchScalarGridSpec(
            num_scalar_prefetch=2,
            grid=(NB,),
            in_specs=[
                pl.BlockSpec((BD, D), lambda b, be, vl: (b, 0)),
                pl.BlockSpec((1, D, 2 * F), lambda b, be, vl: (be[b], 0, 0)),
                pl.BlockSpec((1, 1, 2 * F), lambda b, be, vl: (be[b], 0, 0)),
            ],
            out_specs=pl.BlockSpec((BD, F), lambda b, be, vl: (b, 0)),
        ),
        out_shape=jax.ShapeDtypeStruct((NPAD, F), bf16),
    )(be, valid, xs, w1p, b1p)


def _mlp2_body(be_ref, valid_ref, h_ref, w_ref, b_ref, o_ref):
    b = pl.program_id(0)

    @pl.when(valid_ref[b] == 1)
    def _():
        o_ref[...] = jnp.dot(h_ref[...], w_ref[0],
                             preferred_element_type=f32) + b_ref[0]


def _mlp2(be, valid, hm, w2b, b2r):
    return pl.pallas_call(
        _mlp2_body,
        grid_spec=pltpu.PrefetchScalarGridSpec(
            num_scalar_prefetch=2,
            grid=(NB,),
            in_specs=[
                pl.BlockSpec((BD, F), lambda b, be, vl: (b, 0)),
                pl.BlockSpec((1, F, D), lambda b, be, vl: (be[b], 0, 0)),
                pl.BlockSpec((1, 1, D), lambda b, be, vl: (be[b], 0, 0)),
            ],
            out_specs=pl.BlockSpec((BD, D), lambda b, be, vl: (b, 0)),
        ),
        out_shape=jax.ShapeDtypeStruct((NPAD, D), f32),
    )(be, valid, hm, w2b, b2r)


def _combine_body(h_ref, y0_ref, y1_ref, dw_ref, o_ref):
    w0 = dw_ref[:, 2:3]
    w1 = dw_ref[:, 3:4]
    o_ref[...] = h_ref[...] + w0 * y0_ref[...] + w1 * y1_ref[...]


def _combine(h, y0, y1, dw):
    return pl.pallas_call(
        _combine_body,
        grid=(T // BT,),
        in_specs=[
            pl.BlockSpec((BT, D), lambda i: (i, 0)),
            pl.BlockSpec((BT, D), lambda i: (i, 0)),
            pl.BlockSpec((BT, D), lambda i: (i, 0)),
            pl.BlockSpec((BT, E), lambda i: (i, 0)),
        ],
        out_specs=pl.BlockSpec((BT, D), lambda i: (i, 0)),
        out_shape=jax.ShapeDtypeStruct((T, D), f32),
    )(h, y0, y1, dw)


def kernel(positions, hidden_states, ln1_w, wq, wk, wv, sinks, wo, ln2_w,
           router_w, router_b, w1, b1, w2, b2):
    wqkv = jnp.concatenate([wq, wk, wv], axis=1).astype(bf16)
    half = DH // 2
    inv = 1.0 / (THETA ** (jnp.arange(half, dtype=f32) / half))
    f = positions.astype(f32)[:, None] * inv[None, :]
    cos_t = jnp.concatenate([jnp.cos(f), jnp.cos(f)], axis=1)
    sin_t = jnp.concatenate([-jnp.sin(f), jnp.sin(f)], axis=1)
    ln1 = ln1_w.reshape(1, D)
    ln2 = ln2_w.reshape(1, D)
    rb = router_b.reshape(1, E)
    w1p = jnp.concatenate([w1[:, :, 0::2], w1[:, :, 1::2]], axis=2).astype(bf16)
    b1p = jnp.concatenate([b1[:, 0::2], b1[:, 1::2]], axis=1).reshape(E, 1, 2 * F)
    w2b = w2.astype(bf16)
    b2r = b2.reshape(E, 1, D)

    qkv = _qkv(hidden_states, wqkv, ln1, cos_t, sin_t)
    attn = _attention(qkv, sinks)
    h, x2, logits = _oproj_router(attn, wo.astype(bf16), hidden_states, ln2,
                                  router_w.astype(bf16), rb)
    dw, meta = _route(logits)
    d0 = dw[:, 0].astype(jnp.int32)
    d1 = dw[:, 1].astype(jnp.int32)
    be = meta[:, 0].astype(jnp.int32)
    valid = meta[:, 1].astype(jnp.int32)

    xs = _dispatch_scatter(x2, d0, d1)
    hm = _mlp1(be, valid, xs, w1p, b1p)
    y = _mlp2(be, valid, hm, w2b, b2r)
    y0, y1 = _combine_gather(y, d0, d1)
    return _combine(h, y0, y1, dw)


# speedup vs baseline: 1.8464x; 1.0372x over previous
import functools

import jax
import jax.numpy as jnp
from jax import lax
from jax.experimental import pallas as pl
from jax.experimental.pallas import tpu as pltpu
from jax.experimental.pallas import tpu_sc as plsc

T = 2048; D = 2048; H = 16; HKV = 8; DH = 128; E = 8; F = 2048
ALPHA = 1.702; LIMIT = 7.0; EPS = 1e-6; SCALE = DH ** -0.5
THETA = 10000.0

BT = 256
BD = 128
NB = 40
NPAD = NB * BD
NBPAD = 64

f32 = jnp.float32
bf16 = jnp.bfloat16


NSLOT = H + 2 * HKV


def _qkv_body(x_ref, w_ref, ln_ref, cos_ref, sin_ref, o_ref):
    x = x_ref[...]
    xn = x * lax.rsqrt(jnp.mean(x * x, axis=1, keepdims=True) + EPS)
    xn = xn * ln_ref[...]
    y = jnp.dot(xn.astype(bf16), w_ref[...], preferred_element_type=f32)
    cosb = cos_ref[...]
    sinb = sin_ref[...]
    pieces = []
    for g in range(NSLOT):
        sl = y[:, g * DH:(g + 1) * DH]
        if g < H + HKV:
            sw = jnp.concatenate([sl[:, DH // 2:], sl[:, :DH // 2]], axis=1)
            pieces.append(sl * cosb + sw * sinb)
        else:
            pieces.append(sl)
    o_ref[...] = jnp.concatenate(pieces, axis=1).astype(bf16)


def _qkv(hidden, wqkv, ln1, cos_t, sin_t):
    return pl.pallas_call(
        _qkv_body,
        grid=(T // BT,),
        in_specs=[
            pl.BlockSpec((BT, D), lambda i: (i, 0)),
            pl.BlockSpec((D, NSLOT * DH), lambda i: (0, 0)),
            pl.BlockSpec((1, D), lambda i: (0, 0)),
            pl.BlockSpec((BT, DH), lambda i: (i, 0)),
            pl.BlockSpec((BT, DH), lambda i: (i, 0)),
        ],
        out_specs=pl.BlockSpec((BT, NSLOT * DH), lambda i: (i, 0)),
        out_shape=jax.ShapeDtypeStruct((T, NSLOT * DH), bf16),
    )(hidden, wqkv, ln1, cos_t, sin_t)


BQ = 512
NKC = T // BQ


def _attn_body(q_ref, k_ref, v_ref, sinks_ref, o_ref, sc_ref, acc_ref):
    h = pl.program_id(0)
    i = pl.program_id(1)
    q = q_ref[...]
    for kc in range(NKC):
        @pl.when(kc <= i)
        def _(kc=kc):
            kb = k_ref[kc * BQ:(kc + 1) * BQ, :]
            sc_ref[:, kc * BQ:(kc + 1) * BQ] = lax.dot_general(
                q, kb, (((1,), (1,)), ((), ())),
                preferred_element_type=f32) * SCALE
    s = sc_ref[...]
    row = lax.broadcasted_iota(jnp.int32, (BQ, T), 0) + i * BQ
    col = lax.broadcasted_iota(jnp.int32, (BQ, T), 1)
    s = jnp.where(col <= row, s, -1e30)
    snk = sinks_ref[h]
    m = jnp.maximum(jnp.max(s, axis=1, keepdims=True), snk)
    p = jnp.exp(s - m)
    l = jnp.sum(p, axis=1, keepdims=True) + jnp.exp(snk - m)
    pw = (p / l).astype(bf16)
    acc_ref[...] = jnp.zeros((BQ, DH), f32)
    for kc in range(NKC):
        @pl.when(kc <= i)
        def _(kc=kc):
            acc_ref[...] += jnp.dot(pw[:, kc * BQ:(kc + 1) * BQ],
                                    v_ref[kc * BQ:(kc + 1) * BQ, :],
                                    preferred_element_type=f32)
    o_ref[...] = acc_ref[...].astype(bf16)


def _attention(qkv, sinks):
    return pl.pallas_call(
        _attn_body,
        grid=(H, T // BQ),
        in_specs=[
            pl.BlockSpec((BQ, DH), lambda h, i: (i, h)),
            pl.BlockSpec((T, DH), lambda h, i: (0, H + h // 2)),
            pl.BlockSpec((T, DH), lambda h, i: (0, H + HKV + h // 2)),
            pl.BlockSpec(memory_space=pltpu.SMEM),
        ],
        out_specs=pl.BlockSpec((BQ, DH), lambda h, i: (i, h)),
        out_shape=jax.ShapeDtypeStruct((T, H * DH), bf16),
        scratch_shapes=[pltpu.VMEM((BQ, T), f32), pltpu.VMEM((BQ, DH), f32)],
    )(qkv, qkv, qkv, sinks)


def _oproj_body(attn_ref, wo_ref, hid_ref, ln2_ref, rw_ref, rb_ref,
                h_ref, x2_ref, lg_ref):
    h = hid_ref[...] + jnp.dot(attn_ref[...], wo_ref[...],
                               preferred_element_type=f32)
    h_ref[...] = h
    x2 = h * lax.rsqrt(jnp.mean(h * h, axis=1, keepdims=True) + EPS)
    x2 = x2 * ln2_ref[...]
    x2_ref[...] = x2
    lg_ref[...] = jnp.dot(x2.astype(bf16), rw_ref[...],
                          preferred_element_type=f32) + rb_ref[...]


def _oproj_router(attn, wo, hidden, ln2, rw, rb):
    return pl.pallas_call(
        _oproj_body,
        grid=(T // BT,),
        in_specs=[
            pl.BlockSpec((BT, H * DH), lambda i: (i, 0)),
            pl.BlockSpec((H * DH, D), lambda i: (0, 0)),
            pl.BlockSpec((BT, D), lambda i: (i, 0)),
            pl.BlockSpec((1, D), lambda i: (0, 0)),
            pl.BlockSpec((D, E), lambda i: (0, 0)),
            pl.BlockSpec((1, E), lambda i: (0, 0)),
        ],
        out_specs=[
            pl.BlockSpec((BT, D), lambda i: (i, 0)),
            pl.BlockSpec((BT, D), lambda i: (i, 0)),
            pl.BlockSpec((BT, E), lambda i: (i, 0)),
        ],
        out_shape=[
            jax.ShapeDtypeStruct((T, D), f32),
            jax.ShapeDtypeStruct((T, D), f32),
            jax.ShapeDtypeStruct((T, E), f32),
        ],
    )(attn, wo, hidden, ln2, rw, rb)


def _route_body(lg_ref, dw_ref, meta_ref):
    lg = lg_ref[...]
    jl = lax.broadcasted_iota(jnp.int32, (T, E), 1).astype(f32)
    m1 = jnp.max(lg, axis=1, keepdims=True)
    i1 = jnp.min(jnp.where(lg == m1, jl, 1e9), axis=1, keepdims=True)
    masked = jnp.where(jl == i1, -jnp.inf, lg)
    m2 = jnp.max(masked, axis=1, keepdims=True)
    i2 = jnp.min(jnp.where(masked == m2, jl, 1e9), axis=1, keepdims=True)
    e2 = jnp.exp(m2 - m1)
    w0 = 1.0 / (1.0 + e2)
    w1 = e2 * w0

    oh0 = (jl == i1).astype(f32)
    oh1 = (jl == i2).astype(f32)
    s16 = jnp.concatenate([oh0, oh1], axis=1)
    c = s16
    sh = 1
    while sh < T:
        c = c + jnp.concatenate(
            [jnp.zeros((sh, 2 * E), f32), c[:T - sh]], axis=0)
        sh *= 2
    cexc = c - s16
    ctot = cexc[:, :E] + cexc[:, E:]
    cnt = jnp.sum(s16, axis=0, keepdims=True)
    cnt8 = cnt[:, :E] + cnt[:, E:]
    nb = jnp.floor((cnt8 + (BD - 1.0)) * (1.0 / BD))
    eidx = lax.broadcasted_iota(jnp.int32, (E, E), 0).astype(f32)
    eidx2 = lax.broadcasted_iota(jnp.int32, (E, E), 1).astype(f32)
    mle = (eidx <= eidx2).astype(f32)
    ends = jnp.dot(nb, mle, preferred_element_type=f32) * BD
    pad_off = ends - nb * BD
    rank0 = jnp.sum(oh0 * ctot, axis=1, keepdims=True)
    rank1 = jnp.sum(oh1 * ctot, axis=1, keepdims=True)
    d0 = rank0 + jnp.sum(oh0 * pad_off, axis=1, keepdims=True)
    d1 = rank1 + jnp.sum(oh1 * pad_off, axis=1, keepdims=True)
    dw_ref[...] = jnp.where(jl == 0, d0,
                  jnp.where(jl == 1, d1,
                  jnp.where(jl == 2, w0,
                  jnp.where(jl == 3, w1, 0.0))))

    bi = lax.broadcasted_iota(jnp.int32, (NBPAD, E), 0).astype(f32)
    jb = lax.broadcasted_iota(jnp.int32, (NBPAD, E), 1).astype(f32)
    be = jnp.sum((ends <= BD * bi).astype(f32), axis=1, keepdims=True)
    be = jnp.minimum(be, float(E - 1))
    valid = (BD * bi[:, :1] < ends[:, E - 1:]).astype(f32)
    meta_ref[...] = jnp.where(jb == 0, be, jnp.where(jb == 1, valid, 0.0))


def _route(logits):
    return pl.pallas_call(
        _route_body,
        out_shape=[
            jax.ShapeDtypeStruct((T, E), f32),
            jax.ShapeDtypeStruct((NBPAD, E), f32),
        ],
    )(logits)


_SC_MESH = dict(core_axis_name="c", subcore_axis_name="s")
_NW = 32
_CH = 32
_TPW = T // _NW


def _dispatch_scatter(x2, d0, d1):
    mesh = plsc.VectorSubcoreMesh(**_SC_MESH)

    @functools.partial(
        pl.kernel, mesh=mesh,
        out_type=jax.ShapeDtypeStruct((NPAD, D), f32),
        scratch_types=[
            pltpu.VMEM((_CH, D), f32),
            pltpu.VMEM((_CH,), jnp.int32),
            pltpu.VMEM((_CH,), jnp.int32),
            pltpu.SemaphoreType.DMA,
        ],
    )
    def k(x2_hbm, d0_hbm, d1_hbm, xs_hbm, rows_v, i0_v, i1_v, sem):
        wid = lax.axis_index("s") * 2 + lax.axis_index("c")
        for cnk in range(_TPW // _CH):
            base = wid * _TPW + cnk * _CH
            pltpu.sync_copy(d0_hbm.at[pl.ds(base, _CH)], i0_v)
            pltpu.sync_copy(d1_hbm.at[pl.ds(base, _CH)], i1_v)
            pltpu.sync_copy(x2_hbm.at[pl.ds(base, _CH)], rows_v)
            pltpu.async_copy(rows_v, xs_hbm.at[i0_v], sem).wait()
            pltpu.async_copy(rows_v, xs_hbm.at[i1_v], sem).wait()

    return k(x2, d0, d1)


def _combine_gather(y, d0, d1):
    mesh = plsc.VectorSubcoreMesh(**_SC_MESH)

    @functools.partial(
        pl.kernel, mesh=mesh,
        out_type=(jax.ShapeDtypeStruct((T, D), f32),
                  jax.ShapeDtypeStruct((T, D), f32)),
        scratch_types=[
            pltpu.VMEM((_CH, D), f32),
            pltpu.VMEM((_CH,), jnp.int32),
            pltpu.SemaphoreType.DMA,
        ],
    )
    def k(y_hbm, d0_hbm, d1_hbm, y0_hbm, y1_hbm, rows_v, idx_v, sem):
        wid = lax.axis_index("s") * 2 + lax.axis_index("c")
        for cnk in range(_TPW // _CH):
            base = wid * _TPW + cnk * _CH
            pltpu.sync_copy(d0_hbm.at[pl.ds(base, _CH)], idx_v)
            pltpu.async_copy(y_hbm.at[idx_v], rows_v, sem).wait()
            pltpu.sync_copy(rows_v, y0_hbm.at[pl.ds(base, _CH)])
            pltpu.sync_copy(d1_hbm.at[pl.ds(base, _CH)], idx_v)
            pltpu.async_copy(y_hbm.at[idx_v], rows_v, sem).wait()
            pltpu.sync_copy(rows_v, y1_hbm.at[pl.ds(base, _CH)])

    return k(y, d0, d1)


def _mlp1_body(be_ref, valid_ref, xs_ref, w_ref, b_ref, o_ref):
    b = pl.program_id(0)

    @pl.when(valid_ref[b] == 1)
    def _():
        xb = xs_ref[...].astype(bf16)
        gu = jnp.dot(xb, w_ref[0], preferred_element_type=f32)
        bb = b_ref[0]
        gate = jnp.minimum(gu[:, :F] + bb[:, :F], LIMIT)
        up = jnp.clip(gu[:, F:] + bb[:, F:], -LIMIT, LIMIT)
        glu = gate / (1.0 + jnp.exp(-ALPHA * gate))
        o_ref[...] = ((up + 1.0) * glu).astype(bf16)


def _mlp1(be, valid, xs, w1p, b1p):
    return pl.pallas_call(
        _mlp1_body,
        grid_spec=pltpu.PrefetchScalarGridSpec(
            num_scalar_prefetch=2,
            grid=(NB,),
            in_specs=[
                pl.BlockSpec((BD, D), lambda b, be, vl: (b, 0)),
                pl.BlockSpec((1, D, 2 * F), lambda b, be, vl: (be[b], 0, 0)),
                pl.BlockSpec((1, 1, 2 * F), lambda b, be, vl: (be[b], 0, 0)),
            ],
            out_specs=pl.BlockSpec((BD, F), lambda b, be, vl: (b, 0)),
        ),
        out_shape=jax.ShapeDtypeStruct((NPAD, F), bf16),
    )(be, valid, xs, w1p, b1p)


def _mlp2_body(be_ref, valid_ref, h_ref, w_ref, b_ref, o_ref):
    b = pl.program_id(0)

    @pl.when(valid_ref[b] == 1)
    def _():
        o_ref[...] = jnp.dot(h_ref[...], w_ref[0],
                             preferred_element_type=f32) + b_ref[0]


def _mlp2(be, valid, hm, w2b, b2r):
    return pl.pallas_call(
        _mlp2_body,
        grid_spec=pltpu.PrefetchScalarGridSpec(
            num_scalar_prefetch=2,
            grid=(NB,),
            in_specs=[
                pl.BlockSpec((BD, F), lambda b, be, vl: (b, 0)),
                pl.BlockSpec((1, F, D), lambda b, be, vl: (be[b], 0, 0)),
                pl.BlockSpec((1, 1, D), lambda b, be, vl: (be[b], 0, 0)),
            ],
            out_specs=pl.BlockSpec((BD, D), lambda b, be, vl: (b, 0)),
        ),
        out_shape=jax.ShapeDtypeStruct((NPAD, D), f32),
    )(be, valid, hm, w2b, b2r)


def _combine_body(h_ref, y0_ref, y1_ref, dw_ref, o_ref):
    w0 = dw_ref[:, 2:3]
    w1 = dw_ref[:, 3:4]
    o_ref[...] = h_ref[...] + w0 * y0_ref[...] + w1 * y1_ref[...]


def _combine(h, y0, y1, dw):
    return pl.pallas_call(
        _combine_body,
        grid=(T // BT,),
        in_specs=[
            pl.BlockSpec((BT, D), lambda i: (i, 0)),
            pl.BlockSpec((BT, D), lambda i: (i, 0)),
            pl.BlockSpec((BT, D), lambda i: (i, 0)),
            pl.BlockSpec((BT, E), lambda i: (i, 0)),
        ],
        out_specs=pl.BlockSpec((BT, D), lambda i: (i, 0)),
        out_shape=jax.ShapeDtypeStruct((T, D), f32),
    )(h, y0, y1, dw)


def kernel(positions, hidden_states, ln1_w, wq, wk, wv, sinks, wo, ln2_w,
           router_w, router_b, w1, b1, w2, b2):
    wqkv = jnp.concatenate([wq, wk, wv], axis=1).astype(bf16)
    half = DH // 2
    inv = 1.0 / (THETA ** (jnp.arange(half, dtype=f32) / half))
    f = positions.astype(f32)[:, None] * inv[None, :]
    cos_t = jnp.concatenate([jnp.cos(f), jnp.cos(f)], axis=1)
    sin_t = jnp.concatenate([-jnp.sin(f), jnp.sin(f)], axis=1)
    ln1 = ln1_w.reshape(1, D)
    ln2 = ln2_w.reshape(1, D)
    rb = router_b.reshape(1, E)
    w1p = jnp.concatenate([w1[:, :, 0::2], w1[:, :, 1::2]], axis=2).astype(bf16)
    b1p = jnp.concatenate([b1[:, 0::2], b1[:, 1::2]], axis=1).reshape(E, 1, 2 * F)
    w2b = w2.astype(bf16)
    b2r = b2.reshape(E, 1, D)

    qkv = _qkv(hidden_states, wqkv, ln1, cos_t, sin_t)
    attn = _attention(qkv, sinks)
    h, x2, logits = _oproj_router(attn, wo.astype(bf16), hidden_states, ln2,
                                  router_w.astype(bf16), rb)
    dw, meta = _route(logits)
    d0 = dw[:, 0].astype(jnp.int32)
    d1 = dw[:, 1].astype(jnp.int32)
    be = meta[:, 0].astype(jnp.int32)
    valid = meta[:, 1].astype(jnp.int32)

    xs = _dispatch_scatter(x2, d0, d1)
    hm = _mlp1(be, valid, xs, w1p, b1p)
    y = _mlp2(be, valid, hm, w2b, b2r)
    y0, y1 = _combine_gather(y, d0, d1)
    return _combine(h, y0, y1, dw)


# speedup vs baseline: 7.8980x; 4.2776x over previous
import functools

import jax
import jax.numpy as jnp
from jax import lax
from jax.experimental import pallas as pl
from jax.experimental.pallas import tpu as pltpu
from jax.experimental.pallas import tpu_sc as plsc

T = 2048; D = 2048; H = 16; HKV = 8; DH = 128; E = 8; F = 2048
ALPHA = 1.702; LIMIT = 7.0; EPS = 1e-6; SCALE = DH ** -0.5
THETA = 10000.0

BT = 256
BD = 128
NB = 40
NPAD = NB * BD
NBPAD = 64

f32 = jnp.float32
bf16 = jnp.bfloat16


NSLOT = H + 2 * HKV


def _qkv_body(x_ref, w_ref, ln_ref, cos_ref, sin_ref, o_ref):
    x = x_ref[...]
    xn = x * lax.rsqrt(jnp.mean(x * x, axis=1, keepdims=True) + EPS)
    xn = xn * ln_ref[...]
    y = jnp.dot(xn.astype(bf16), w_ref[...], preferred_element_type=f32)
    cosb = cos_ref[...]
    sinb = sin_ref[...]
    pieces = []
    for g in range(NSLOT):
        sl = y[:, g * DH:(g + 1) * DH]
        if g < H + HKV:
            sw = jnp.concatenate([sl[:, DH // 2:], sl[:, :DH // 2]], axis=1)
            pieces.append(sl * cosb + sw * sinb)
        else:
            pieces.append(sl)
    o_ref[...] = jnp.concatenate(pieces, axis=1).astype(bf16)


def _qkv(hidden, wqkv, ln1, cos_t, sin_t):
    return pl.pallas_call(
        _qkv_body,
        grid=(T // BT,),
        in_specs=[
            pl.BlockSpec((BT, D), lambda i: (i, 0)),
            pl.BlockSpec((D, NSLOT * DH), lambda i: (0, 0)),
            pl.BlockSpec((1, D), lambda i: (0, 0)),
            pl.BlockSpec((BT, DH), lambda i: (i, 0)),
            pl.BlockSpec((BT, DH), lambda i: (i, 0)),
        ],
        out_specs=pl.BlockSpec((BT, NSLOT * DH), lambda i: (i, 0)),
        out_shape=jax.ShapeDtypeStruct((T, NSLOT * DH), bf16),
    )(hidden, wqkv, ln1, cos_t, sin_t)


BQ = 512
NKC = T // BQ


def _attn_body(q_ref, k_ref, v_ref, sinks_ref, o_ref, sc_ref, acc_ref):
    h = pl.program_id(0)
    i = pl.program_id(1)
    q = q_ref[...]
    for kc in range(NKC):
        @pl.when(kc <= i)
        def _(kc=kc):
            kb = k_ref[kc * BQ:(kc + 1) * BQ, :]
            sc_ref[:, kc * BQ:(kc + 1) * BQ] = lax.dot_general(
                q, kb, (((1,), (1,)), ((), ())),
                preferred_element_type=f32) * SCALE
    s = sc_ref[...]
    row = lax.broadcasted_iota(jnp.int32, (BQ, T), 0) + i * BQ
    col = lax.broadcasted_iota(jnp.int32, (BQ, T), 1)
    s = jnp.where(col <= row, s, -1e30)
    snk = sinks_ref[h]
    m = jnp.maximum(jnp.max(s, axis=1, keepdims=True), snk)
    p = jnp.exp(s - m)
    l = jnp.sum(p, axis=1, keepdims=True) + jnp.exp(snk - m)
    pw = (p / l).astype(bf16)
    acc_ref[...] = jnp.zeros((BQ, DH), f32)
    for kc in range(NKC):
        @pl.when(kc <= i)
        def _(kc=kc):
            acc_ref[...] += jnp.dot(pw[:, kc * BQ:(kc + 1) * BQ],
                                    v_ref[kc * BQ:(kc + 1) * BQ, :],
                                    preferred_element_type=f32)
    o_ref[...] = acc_ref[...].astype(bf16)


def _attention(qkv, sinks):
    return pl.pallas_call(
        _attn_body,
        grid=(H, T // BQ),
        in_specs=[
            pl.BlockSpec((BQ, DH), lambda h, i: (i, h)),
            pl.BlockSpec((T, DH), lambda h, i: (0, H + h // 2)),
            pl.BlockSpec((T, DH), lambda h, i: (0, H + HKV + h // 2)),
            pl.BlockSpec(memory_space=pltpu.SMEM),
        ],
        out_specs=pl.BlockSpec((BQ, DH), lambda h, i: (i, h)),
        out_shape=jax.ShapeDtypeStruct((T, H * DH), bf16),
        scratch_shapes=[pltpu.VMEM((BQ, T), f32), pltpu.VMEM((BQ, DH), f32)],
    )(qkv, qkv, qkv, sinks)


def _oproj_body(attn_ref, wo_ref, hid_ref, ln2_ref, rw_ref, rb_ref,
                h_ref, x2_ref, lg_ref):
    h = hid_ref[...] + jnp.dot(attn_ref[...], wo_ref[...],
                               preferred_element_type=f32)
    h_ref[...] = h
    x2 = h * lax.rsqrt(jnp.mean(h * h, axis=1, keepdims=True) + EPS)
    x2 = x2 * ln2_ref[...]
    x2_ref[...] = x2
    lg_ref[...] = jnp.dot(x2.astype(bf16), rw_ref[...],
                          preferred_element_type=f32) + rb_ref[...]


def _oproj_router(attn, wo, hidden, ln2, rw, rb):
    return pl.pallas_call(
        _oproj_body,
        grid=(T // BT,),
        in_specs=[
            pl.BlockSpec((BT, H * DH), lambda i: (i, 0)),
            pl.BlockSpec((H * DH, D), lambda i: (0, 0)),
            pl.BlockSpec((BT, D), lambda i: (i, 0)),
            pl.BlockSpec((1, D), lambda i: (0, 0)),
            pl.BlockSpec((D, E), lambda i: (0, 0)),
            pl.BlockSpec((1, E), lambda i: (0, 0)),
        ],
        out_specs=[
            pl.BlockSpec((BT, D), lambda i: (i, 0)),
            pl.BlockSpec((BT, D), lambda i: (i, 0)),
            pl.BlockSpec((BT, E), lambda i: (i, 0)),
        ],
        out_shape=[
            jax.ShapeDtypeStruct((T, D), f32),
            jax.ShapeDtypeStruct((T, D), f32),
            jax.ShapeDtypeStruct((T, E), f32),
        ],
    )(attn, wo, hidden, ln2, rw, rb)


def _route_body(lg_ref, dw_ref, meta_ref):
    lg = lg_ref[...]
    jl = lax.broadcasted_iota(jnp.int32, (T, E), 1).astype(f32)
    m1 = jnp.max(lg, axis=1, keepdims=True)
    i1 = jnp.min(jnp.where(lg == m1, jl, 1e9), axis=1, keepdims=True)
    masked = jnp.where(jl == i1, -jnp.inf, lg)
    m2 = jnp.max(masked, axis=1, keepdims=True)
    i2 = jnp.min(jnp.where(masked == m2, jl, 1e9), axis=1, keepdims=True)
    e2 = jnp.exp(m2 - m1)
    w0 = 1.0 / (1.0 + e2)
    w1 = e2 * w0

    oh0 = (jl == i1).astype(f32)
    oh1 = (jl == i2).astype(f32)
    s16 = jnp.concatenate([oh0, oh1], axis=1)
    c = s16
    sh = 1
    while sh < T:
        c = c + jnp.concatenate(
            [jnp.zeros((sh, 2 * E), f32), c[:T - sh]], axis=0)
        sh *= 2
    cexc = c - s16
    ctot = cexc[:, :E] + cexc[:, E:]
    cnt = jnp.sum(s16, axis=0, keepdims=True)
    cnt8 = cnt[:, :E] + cnt[:, E:]
    nb = jnp.floor((cnt8 + (BD - 1.0)) * (1.0 / BD))
    eidx = lax.broadcasted_iota(jnp.int32, (E, E), 0).astype(f32)
    eidx2 = lax.broadcasted_iota(jnp.int32, (E, E), 1).astype(f32)
    mle = (eidx <= eidx2).astype(f32)
    ends = jnp.dot(nb, mle, preferred_element_type=f32) * BD
    pad_off = ends - nb * BD
    rank0 = jnp.sum(oh0 * ctot, axis=1, keepdims=True)
    rank1 = jnp.sum(oh1 * ctot, axis=1, keepdims=True)
    d0 = rank0 + jnp.sum(oh0 * pad_off, axis=1, keepdims=True)
    d1 = rank1 + jnp.sum(oh1 * pad_off, axis=1, keepdims=True)
    dw_ref[...] = jnp.where(jl == 0, d0,
                  jnp.where(jl == 1, d1,
                  jnp.where(jl == 2, w0,
                  jnp.where(jl == 3, w1, 0.0))))

    bi = lax.broadcasted_iota(jnp.int32, (NBPAD, E), 0).astype(f32)
    jb = lax.broadcasted_iota(jnp.int32, (NBPAD, E), 1).astype(f32)
    be = jnp.sum((ends <= BD * bi).astype(f32), axis=1, keepdims=True)
    be = jnp.minimum(be, float(E - 1))
    valid = (BD * bi[:, :1] < ends[:, E - 1:]).astype(f32)
    meta_ref[...] = jnp.where(jb == 0, be, jnp.where(jb == 1, valid, 0.0))


def _route(logits):
    return pl.pallas_call(
        _route_body,
        out_shape=[
            jax.ShapeDtypeStruct((T, E), f32),
            jax.ShapeDtypeStruct((NBPAD, E), f32),
        ],
    )(logits)


_SC_MESH = dict(core_axis_name="c", subcore_axis_name="s")
_NW = 32
_CH = 32
_TPW = T // _NW


def _dispatch_scatter(x2, d0, d1):
    mesh = plsc.VectorSubcoreMesh(**_SC_MESH)

    @functools.partial(
        pl.kernel, mesh=mesh,
        out_type=jax.ShapeDtypeStruct((NPAD, D), f32),
        scratch_types=[
            pltpu.VMEM((_CH, D), f32),
            pltpu.VMEM((_CH,), jnp.int32),
            pltpu.VMEM((_CH,), jnp.int32),
            pltpu.SemaphoreType.DMA,
        ],
    )
    def k(x2_hbm, d0_hbm, d1_hbm, xs_hbm, rows_v, i0_v, i1_v, sem):
        wid = lax.axis_index("s") * 2 + lax.axis_index("c")
        for cnk in range(_TPW // _CH):
            base = wid * _TPW + cnk * _CH
            pltpu.sync_copy(d0_hbm.at[pl.ds(base, _CH)], i0_v)
            pltpu.sync_copy(d1_hbm.at[pl.ds(base, _CH)], i1_v)
            pltpu.sync_copy(x2_hbm.at[pl.ds(base, _CH)], rows_v)
            pltpu.async_copy(rows_v, xs_hbm.at[i0_v], sem).wait()
            pltpu.async_copy(rows_v, xs_hbm.at[i1_v], sem).wait()

    return k(x2, d0, d1)


def _combine_gather(y, d0, d1):
    mesh = plsc.VectorSubcoreMesh(**_SC_MESH)

    @functools.partial(
        pl.kernel, mesh=mesh,
        out_type=(jax.ShapeDtypeStruct((T, D), f32),
                  jax.ShapeDtypeStruct((T, D), f32)),
        scratch_types=[
            pltpu.VMEM((_CH, D), f32),
            pltpu.VMEM((_CH,), jnp.int32),
            pltpu.SemaphoreType.DMA,
        ],
    )
    def k(y_hbm, d0_hbm, d1_hbm, y0_hbm, y1_hbm, rows_v, idx_v, sem):
        wid = lax.axis_index("s") * 2 + lax.axis_index("c")
        for cnk in range(_TPW // _CH):
            base = wid * _TPW + cnk * _CH
            pltpu.sync_copy(d0_hbm.at[pl.ds(base, _CH)], idx_v)
            pltpu.async_copy(y_hbm.at[idx_v], rows_v, sem).wait()
            pltpu.sync_copy(rows_v, y0_hbm.at[pl.ds(base, _CH)])
            pltpu.sync_copy(d1_hbm.at[pl.ds(base, _CH)], idx_v)
            pltpu.async_copy(y_hbm.at[idx_v], rows_v, sem).wait()
            pltpu.sync_copy(rows_v, y1_hbm.at[pl.ds(base, _CH)])

    return k(y, d0, d1)


def _mlp1_body(be_ref, valid_ref, xs_ref, w_ref, b_ref, o_ref):
    b = pl.program_id(0)

    @pl.when(valid_ref[b] == 1)
    def _():
        xb = xs_ref[...].astype(bf16)
        gu = jnp.dot(xb, w_ref[0], preferred_element_type=f32)
        bb = b_ref[0]
        gate = jnp.minimum(gu[:, :F] + bb[:, :F], LIMIT)
        up = jnp.clip(gu[:, F:] + bb[:, F:], -LIMIT, LIMIT)
        glu = gate / (1.0 + jnp.exp(-ALPHA * gate))
        o_ref[...] = ((up + 1.0) * glu).astype(bf16)


def _mlp1(be, valid, xs, w1p, b1p):
    return pl.pallas_call(
        _mlp1_body,
        grid_spec=pltpu.PrefetchScalarGridSpec(
            num_scalar_prefetch=2,
            grid=(NB,),
            in_specs=[
                pl.BlockSpec((BD, D), lambda b, be, vl: (b, 0)),
                pl.BlockSpec((1, D, 2 * F), lambda b, be, vl: (be[b], 0, 0)),
                pl.BlockSpec((1, 1, 2 * F), lambda b, be, vl: (be[b], 0, 0)),
            ],
            out_specs=pl.BlockSpec((BD, F), lambda b, be, vl: (b, 0)),
        ),
        out_shape=jax.ShapeDtypeStruct((NPAD, F), bf16),
    )(be, valid, xs, w1p, b1p)


def _mlp2_body(be_ref, valid_ref, h_ref, w_ref, b_ref, o_ref):
    b = pl.program_id(0)

    @pl.when(valid_ref[b] == 1)
    def _():
        o_ref[...] = jnp.dot(h_ref[...], w_ref[0],
                             preferred_element_type=f32) + b_ref[0]


def _mlp2(be, valid, hm, w2b, b2r):
    return pl.pallas_call(
        _mlp2_body,
        grid_spec=pltpu.PrefetchScalarGridSpec(
            num_scalar_prefetch=2,
            grid=(NB,),
            in_specs=[
                pl.BlockSpec((BD, F), lambda b, be, vl: (b, 0)),
                pl.BlockSpec((1, F, D), lambda b, be, vl: (be[b], 0, 0)),
                pl.BlockSpec((1, 1, D), lambda b, be, vl: (be[b], 0, 0)),
            ],
            out_specs=pl.BlockSpec((BD, D), lambda b, be, vl: (b, 0)),
        ),
        out_shape=jax.ShapeDtypeStruct((NPAD, D), f32),
    )(be, valid, hm, w2b, b2r)


def _combine_body(h_ref, y0_ref, y1_ref, dw_ref, o_ref):
    w0 = dw_ref[:, 2:3]
    w1 = dw_ref[:, 3:4]
    o_ref[...] = h_ref[...] + w0 * y0_ref[...] + w1 * y1_ref[...]


def _combine(h, y0, y1, dw):
    return pl.pallas_call(
        _combine_body,
        grid=(T // BT,),
        in_specs=[
            pl.BlockSpec((BT, D), lambda i: (i, 0)),
            pl.BlockSpec((BT, D), lambda i: (i, 0)),
            pl.BlockSpec((BT, D), lambda i: (i, 0)),
            pl.BlockSpec((BT, E), lambda i: (i, 0)),
        ],
        out_specs=pl.BlockSpec((BT, D), lambda i: (i, 0)),
        out_shape=jax.ShapeDtypeStruct((T, D), f32),
    )(h, y0, y1, dw)


def kernel(positions, hidden_states, ln1_w, wq, wk, wv, sinks, wo, ln2_w,
           router_w, router_b, w1, b1, w2, b2):
    wqkv = jnp.concatenate([wq, wk, wv], axis=1).astype(bf16)
    half = DH // 2
    inv = 1.0 / (THETA ** (jnp.arange(half, dtype=f32) / half))
    f = positions.astype(f32)[:, None] * inv[None, :]
    cos_t = jnp.concatenate([jnp.cos(f), jnp.cos(f)], axis=1)
    sin_t = jnp.concatenate([-jnp.sin(f), jnp.sin(f)], axis=1)
    ln1 = ln1_w.reshape(1, D)
    ln2 = ln2_w.reshape(1, D)
    rb = router_b.reshape(1, E)
    w1p = (w1.reshape(E, D, F, 2).transpose(0, 1, 3, 2)
           .reshape(E, D, 2 * F).astype(bf16))
    b1p = jnp.concatenate([b1[:, 0::2], b1[:, 1::2]], axis=1).reshape(E, 1, 2 * F)
    w2b = w2.astype(bf16)
    b2r = b2.reshape(E, 1, D)

    qkv = _qkv(hidden_states, wqkv, ln1, cos_t, sin_t)
    attn = _attention(qkv, sinks)
    h, x2, logits = _oproj_router(attn, wo.astype(bf16), hidden_states, ln2,
                                  router_w.astype(bf16), rb)
    dw, meta = _route(logits)
    d0 = dw[:, 0].astype(jnp.int32)
    d1 = dw[:, 1].astype(jnp.int32)
    be = meta[:, 0].astype(jnp.int32)
    valid = meta[:, 1].astype(jnp.int32)

    xs = _dispatch_scatter(x2, d0, d1)
    hm = _mlp1(be, valid, xs, w1p, b1p)
    y = _mlp2(be, valid, hm, w2b, b2r)
    y0, y1 = _combine_gather(y, d0, d1)
    return _combine(h, y0, y1, dw)
